# baseline (speedup 1.0000x reference)
import functools

import jax
import jax.numpy as jnp
from jax import lax
from jax.experimental import pallas as pl
from jax.experimental.pallas import tpu as pltpu
from jax.experimental.pallas import tpu_sc as plsc

_SEGMENTS = [-2.0, -1.5, -1.0, -0.5, 0.0, 0.5, 1.0, 1.5, 2.0]
_LAST = float(_SEGMENTS[-1])
_LO = len(_SEGMENTS)
_HI = len(_SEGMENTS) + 1

_N = 4194304
_NC = 2
_NS = 16
_LANES = 16
_NW = _NC * _NS
_PER_W = _N // _NW
_CHUNK = 8192
_NBUF = 12
_NCHUNK = _PER_W // _CHUNK
_PRELAG = 2


def _sc_body(x_hbm, out_hbm, *scratch):
    wid = lax.axis_index("s") * _NC + lax.axis_index("c")
    base = wid * _PER_W

    bufs = scratch[:_NBUF]
    in_sems = scratch[_NBUF:2 * _NBUF]
    out_sems = scratch[2 * _NBUF:3 * _NBUF]

    lo = jnp.full((_LANES,), _LO, jnp.int32)
    hi = jnp.full((_LANES,), _HI, jnp.int32)

    def start_in(c):
        return pltpu.async_copy(
            x_hbm.at[pl.ds(base + c * _CHUNK, _CHUNK)],
            bufs[c % _NBUF], in_sems[c % _NBUF])

    def start_out(c):
        return pltpu.async_copy(
            bufs[c % _NBUF],
            out_hbm.at[pl.ds(base + c * _CHUNK, _CHUNK)],
            out_sems[c % _NBUF])

    h_in = [start_in(c) for c in range(min(_NBUF, _NCHUNK))]
    h_out = [None] * _NBUF
    for c in range(_NCHUNK):
        b = c % _NBUF
        h_in[b].wait()
        buf = bufs[b]

        @plsc.parallel_loop(0, _CHUNK, step=_LANES, unroll=8)
        def _(i):
            x = buf[pl.ds(i, _LANES)]
            r = jnp.where(x < _LAST, lo, hi)
            buf[pl.ds(i, _LANES)] = plsc.bitcast(r, jnp.float32)

        h_out[b] = start_out(c)
        j = c - _PRELAG
        if j >= 0 and j + _NBUF < _NCHUNK:
            bj = j % _NBUF
            h_out[bj].wait()
            h_out[bj] = None
            h_in[bj] = start_in(j + _NBUF)
    for b in range(_NBUF):
        if h_out[b] is not None:
            h_out[b].wait()


_sc_call = functools.partial(
    pl.kernel,
    mesh=plsc.VectorSubcoreMesh(core_axis_name="c", subcore_axis_name="s"),
    out_type=jax.ShapeDtypeStruct((_N,), jnp.float32),
    scratch_types=(
        [pltpu.VMEM((_CHUNK,), jnp.float32) for _ in range(_NBUF)]
        + [pltpu.SemaphoreType.DMA for _ in range(2 * _NBUF)]
    ),
    compiler_params=pltpu.CompilerParams(
        skip_device_barrier=True, needs_layout_passes=False),
)(_sc_body)


def kernel(inputs):
    flat = inputs.reshape(_N)
    out = _sc_call(flat)
    return lax.bitcast_convert_type(out, jnp.int32).reshape(_N, 1)


# speedup vs baseline: 1.3652x; 1.3652x over previous
import functools

import jax
import jax.numpy as jnp
from jax import lax
from jax.experimental import pallas as pl
from jax.experimental.pallas import tpu as pltpu
from jax.experimental.pallas import tpu_sc as plsc

_SEGMENTS = [-2.0, -1.5, -1.0, -0.5, 0.0, 0.5, 1.0, 1.5, 2.0]
_LAST = float(_SEGMENTS[-1])
_LO = len(_SEGMENTS)
_HI = len(_SEGMENTS) + 1

_N = 4194304
_NC = 2
_NS = 16
_LANES = 16
_NW = _NC * _NS
_PER_W = _N // _NW
_CHUNK = 8192
_NBUF = 7
_NCHUNK = _PER_W // _CHUNK


def _sc_body(x_hbm, out_hbm, *scratch):
    wid = lax.axis_index("s") * _NC + lax.axis_index("c")
    base = wid * _PER_W

    in_bufs = scratch[:_NBUF]
    out_bufs = scratch[_NBUF:2 * _NBUF]
    in_sems = scratch[2 * _NBUF:3 * _NBUF]
    out_sems = scratch[3 * _NBUF:4 * _NBUF]

    lo = jnp.full((_LANES,), _LO, jnp.int32)
    hi = jnp.full((_LANES,), _HI, jnp.int32)

    def start_in(c):
        return pltpu.async_copy(
            x_hbm.at[pl.ds(base + c * _CHUNK, _CHUNK)],
            in_bufs[c % _NBUF], in_sems[c % _NBUF])

    def start_out(c):
        return pltpu.async_copy(
            out_bufs[c % _NBUF],
            out_hbm.at[pl.ds(base + c * _CHUNK, _CHUNK)],
            out_sems[c % _NBUF])

    h_in = [start_in(c) for c in range(_NBUF)]
    h_out = [None] * _NBUF
    for c in range(_NCHUNK):
        b = c % _NBUF
        if h_out[b] is not None:
            h_out[b].wait()
        h_in[b].wait()
        in_b, out_b = in_bufs[b], out_bufs[b]

        @plsc.parallel_loop(0, _CHUNK, step=_LANES, unroll=8)
        def _(i):
            x = in_b[pl.ds(i, _LANES)]
            out_b[pl.ds(i, _LANES)] = jnp.where(x < _LAST, lo, hi)

        h_out[b] = start_out(c)
        if c + _NBUF < _NCHUNK:
            h_in[b] = start_in(c + _NBUF)
    for b in range(_NBUF):
        h_out[b].wait()


_sc_call = functools.partial(
    pl.kernel,
    mesh=plsc.VectorSubcoreMesh(core_axis_name="c", subcore_axis_name="s"),
    out_type=jax.ShapeDtypeStruct((_N,), jnp.int32),
    scratch_types=(
        [pltpu.VMEM((_CHUNK,), jnp.float32) for _ in range(_NBUF)]
        + [pltpu.VMEM((_CHUNK,), jnp.int32) for _ in range(_NBUF)]
        + [pltpu.SemaphoreType.DMA for _ in range(2 * _NBUF)]
    ),
    compiler_params=pltpu.CompilerParams(skip_device_barrier=True),
)(_sc_body)


def kernel(inputs):
    flat = inputs.reshape(_N)
    out = _sc_call(flat)
    return out.reshape(_N, 1)


# speedup vs baseline: 1.3743x; 1.0067x over previous
import functools

import jax
import jax.numpy as jnp
from jax import lax
from jax.experimental import pallas as pl
from jax.experimental.pallas import tpu as pltpu
from jax.experimental.pallas import tpu_sc as plsc

_SEGMENTS = [-2.0, -1.5, -1.0, -0.5, 0.0, 0.5, 1.0, 1.5, 2.0]
_LAST = float(_SEGMENTS[-1])
_LO = len(_SEGMENTS)
_HI = len(_SEGMENTS) + 1

_N = 4194304
_NC = 2
_NS = 16
_LANES = 16
_NW = _NC * _NS
_PER_W = _N // _NW
_CHUNK = 8192
_NIB = 9
_NOB = 5
_NBUF = 7
_NCHUNK = _PER_W // _CHUNK


def _sc_body(x_hbm, out_hbm, *scratch):
    wid = lax.axis_index("s") * _NC + lax.axis_index("c")
    base = wid * _PER_W

    in_bufs = scratch[:_NIB]
    out_bufs = scratch[_NIB:_NIB + _NOB]
    in_sems = scratch[_NIB + _NOB:2 * _NIB + _NOB]
    out_sems = scratch[2 * _NIB + _NOB:]

    lo = jnp.full((_LANES,), _LO, jnp.int32)
    hi = jnp.full((_LANES,), _HI, jnp.int32)

    def start_in(c):
        return pltpu.async_copy(
            x_hbm.at[pl.ds(base + c * _CHUNK, _CHUNK)],
            in_bufs[c % _NIB], in_sems[c % _NIB])

    def start_out(c):
        return pltpu.async_copy(
            out_bufs[c % _NOB],
            out_hbm.at[pl.ds(base + c * _CHUNK, _CHUNK)],
            out_sems[c % _NOB])

    h_in = [start_in(c) for c in range(_NIB)]
    h_out = [None] * _NOB
    for c in range(_NCHUNK):
        bi = c % _NIB
        bo = c % _NOB
        if h_out[bo] is not None:
            h_out[bo].wait()
        h_in[bi].wait()
        in_b, out_b = in_bufs[bi], out_bufs[bo]

        @plsc.parallel_loop(0, _CHUNK, step=_LANES, unroll=8)
        def _(i):
            x = in_b[pl.ds(i, _LANES)]
            out_b[pl.ds(i, _LANES)] = jnp.where(x < _LAST, lo, hi)

        h_out[bo] = start_out(c)
        if c + _NIB < _NCHUNK:
            h_in[bi] = start_in(c + _NIB)
    for b in range(_NOB):
        h_out[b].wait()


_sc_call = functools.partial(
    pl.kernel,
    mesh=plsc.VectorSubcoreMesh(core_axis_name="c", subcore_axis_name="s"),
    out_type=jax.ShapeDtypeStruct((_N,), jnp.int32),
    scratch_types=(
        [pltpu.VMEM((_CHUNK,), jnp.float32) for _ in range(_NIB)]
        + [pltpu.VMEM((_CHUNK,), jnp.int32) for _ in range(_NOB)]
        + [pltpu.SemaphoreType.DMA for _ in range(_NIB + _NOB)]
    ),
    compiler_params=pltpu.CompilerParams(skip_device_barrier=True),
)(_sc_body)


def kernel(inputs):
    flat = inputs.reshape(_N)
    out = _sc_call(flat)
    return out.reshape(_N, 1)
